# baseline (speedup 1.0000x reference)
import functools

import jax
import jax.numpy as jnp
from jax import lax
from jax.experimental import pallas as pl
from jax.experimental.pallas import tpu as pltpu
from jax.experimental.pallas import tpu_sc as plsc

_NC = 2
_NS = 16
_NW = _NC * _NS
_CHUNK = 128


def _enc_nodes_body(x_ref, wn_ref, bn_ref, t_ref, wt_ref, bt_ref,
                    h_ref, hmax_ref):
    x0 = jnp.dot(x_ref[...], wn_ref[...],
                 preferred_element_type=jnp.float32) + bn_ref[...]
    tw = t_ref[...] * wt_ref[...] + bt_ref[...]
    h = x0 * tw
    h_ref[...] = h
    hmax_ref[...] = jnp.max(h, axis=0, keepdims=True)


def _enc_nodes(x, W_node, b_node, time_weights, W_time, b_time):
    n = x.shape[0]
    return pl.pallas_call(
        _enc_nodes_body,
        out_shape=[
            jax.ShapeDtypeStruct((n, 16), jnp.float32),
            jax.ShapeDtypeStruct((1, 16), jnp.float32),
        ],
    )(x, W_node, b_node.reshape(1, 16), time_weights,
      W_time, b_time.reshape(1, 16))


def _enc_edges_body(a_ref, w_ref, b_ref, ea_ref, emax_ref, zbuf, sem_out):
    i = pl.program_id(0)
    ng = pl.num_programs(0)
    blk = a_ref.shape[0]

    def out_cp(j):
        return pltpu.make_async_copy(zbuf.at[j % 2],
                                     ea_ref.at[pl.ds(j * blk, blk)], sem_out)

    @pl.when(i >= 2)
    def _():
        out_cp(i - 2).wait()

    z = jnp.dot(a_ref[...], w_ref[...],
                preferred_element_type=jnp.float32) + b_ref[...]
    zbuf[i % 2] = z
    out_cp(i).start()
    bm = jnp.max(z, axis=0, keepdims=True)

    @pl.when(i == 0)
    def _():
        emax_ref[...] = bm

    @pl.when(i > 0)
    def _():
        emax_ref[...] = jnp.maximum(emax_ref[...], bm)

    @pl.when(i == ng - 1)
    def _():
        out_cp(i - 1).wait()
        out_cp(i).wait()


def _enc_edges(edge_attr, W_edge, b_edge):
    e = edge_attr.shape[0]
    blk = 5000
    grid = e // blk
    ea_lin, emax_p = pl.pallas_call(
        _enc_edges_body,
        grid=(grid,),
        in_specs=[
            pl.BlockSpec((blk, 16), lambda i: (i, 0)),
            pl.BlockSpec((16, 16), lambda i: (0, 0)),
            pl.BlockSpec((1, 16), lambda i: (0, 0)),
        ],
        out_specs=[
            pl.BlockSpec(memory_space=pl.ANY),
            pl.BlockSpec((1, 16), lambda i: (0, 0)),
        ],
        out_shape=[
            jax.ShapeDtypeStruct((e, 16), jnp.float32),
            jax.ShapeDtypeStruct((1, 16), jnp.float32),
        ],
        scratch_shapes=[
            pltpu.VMEM((2, blk, 16), jnp.float32),
            pltpu.SemaphoreType.DMA,
        ],
    )(edge_attr, W_edge, b_edge.reshape(1, 16))
    return ea_lin, emax_p


_G = 5
_GE = _G * _CHUNK


def _edge_pass(h, ea_lin, src, dst, hmax, eamax_p):
    n = h.shape[0]
    e = ea_lin.shape[0]
    n_chunks = e // _CHUNK
    n_groups = n_chunks // _G
    iters = pl.cdiv(n_groups, _NW)
    pairs = pl.cdiv(iters, 2)
    npad = ((n + 8 * _NS - 1) // (8 * _NS)) * (8 * _NS)
    zrows = npad // _NS

    src2 = src.reshape(n_chunks, _CHUNK)
    dst2 = dst.reshape(n_chunks, _CHUNK)

    mesh = plsc.VectorSubcoreMesh(core_axis_name="c", subcore_axis_name="s")

    @functools.partial(
        pl.kernel,
        out_type=jax.ShapeDtypeStruct((_NC * npad, 32), jnp.float32),
        mesh=mesh,
        scratch_types=[
            pltpu.VMEM((4, _G, _CHUNK), jnp.int32),
            pltpu.VMEM((4, _G, _CHUNK), jnp.int32),
            pltpu.VMEM((2, _GE, 16), jnp.float32),
            pltpu.VMEM((2, _GE, 16), jnp.float32),
            pltpu.VMEM((2, _GE, 32), jnp.float32),
            pltpu.VMEM((16,), jnp.float32),
            pltpu.VMEM((16,), jnp.float32),
            pltpu.VMEM((zrows, 32), jnp.float32),
            pltpu.VMEM_SHARED((npad, 32), jnp.float32),
            pltpu.SemaphoreType.DMA,
            pltpu.SemaphoreType.DMA,
            pltpu.SemaphoreType.DMA,
            pltpu.SemaphoreType.DMA,
            pltpu.SemaphoreType.DMA,
        ],
        compiler_params=pltpu.CompilerParams(use_tc_tiling_on_sc=False),
    )
    def k(h_hbm, ea_hbm, src_hbm, dst_hbm, hmax_hbm, eamax_hbm, out_hbm,
          sidx, didx, hrows, earows, tp, hmv, emv, zbuf, acc,
          sem_s, sem_d, sem_e, sem_g, sem_sc):
        cid = lax.axis_index("c")
        sid = lax.axis_index("s")
        wid = sid * _NC + cid

        zero16 = jnp.zeros((16,), jnp.float32)

        @plsc.parallel_loop(0, zrows, step=1, unroll=4)
        def _(i):
            zbuf[i, pl.ds(0, 16)] = zero16
            zbuf[i, pl.ds(16, 16)] = zero16

        pltpu.sync_copy(zbuf, acc.at[pl.ds(sid * zrows, zrows)])
        pltpu.sync_copy(hmax_hbm, hmv)
        pltpu.sync_copy(eamax_hbm, emv)
        shvec = jnp.maximum(hmv[...] + emv[...], 0.0) + 1e-7
        plsc.subcore_barrier()

        def issue_loads(g, s4, q):
            pltpu.async_copy(src_hbm.at[pl.ds(g * _G, _G)], sidx.at[s4],
                             sem_s)
            pltpu.async_copy(dst_hbm.at[pl.ds(g * _G, _G)], didx.at[s4],
                             sem_d)
            pltpu.async_copy(ea_hbm.at[pl.ds(g * _GE, _GE)], earows.at[q],
                             sem_e)

        def drain_idx(g, s4):
            pltpu.make_async_copy(src_hbm.at[pl.ds(g * _G, _G)],
                                  sidx.at[s4], sem_s).wait()
            pltpu.make_async_copy(dst_hbm.at[pl.ds(g * _G, _G)],
                                  didx.at[s4], sem_d).wait()

        def issue_gathers(s4, q):
            for b in range(_G):
                pltpu.async_copy(h_hbm.at[sidx.at[s4, b]],
                                 hrows.at[q, pl.ds(b * _CHUNK, _CHUNK)],
                                 sem_g)

        def drain_gathers(s4, q):
            for b in range(_G):
                pltpu.make_async_copy(h_hbm.at[sidx.at[s4, b]],
                                      hrows.at[q, pl.ds(b * _CHUNK, _CHUNK)],
                                      sem_g).wait()

        def issue_scatters(s4, q):
            for b in range(_G):
                pltpu.async_copy(tp.at[q, pl.ds(b * _CHUNK, _CHUNK)],
                                 acc.at[didx.at[s4, b]], sem_sc, add=True)

        def drain_scatters(s4, q):
            for b in range(_G):
                pltpu.make_async_copy(tp.at[q, pl.ds(b * _CHUNK, _CHUNK)],
                                      acc.at[didx.at[s4, b]], sem_sc).wait()

        issue_loads(wid, 0, 0)
        issue_loads(_NW + wid, 1, 1)
        drain_idx(wid, 0)
        issue_gathers(0, 0)

        def quad_body(j4, _):
            for qq in range(4):
                q = qq % 2
                i = j4 * 4 + qq
                g = i * _NW + wid
                g1 = g + _NW
                g2 = g + 2 * _NW

                @pl.when(g1 < n_groups)
                def _():
                    drain_idx(g1, (qq + 1) % 4)
                    issue_gathers((qq + 1) % 4, 1 - q)

                @pl.when(g < n_groups)
                def _():
                    pltpu.make_async_copy(
                        ea_hbm.at[pl.ds(g * _GE, _GE)], earows.at[q],
                        sem_e).wait()
                    drain_gathers(qq, q)

                @pl.when((i >= 2) & (g - 2 * _NW < n_groups))
                def _():
                    drain_scatters((qq + 2) % 4, q)

                @pl.when(g < n_groups)
                def _():
                    @plsc.parallel_loop(0, _GE, step=1, unroll=4)
                    def _(r):
                        msg = jnp.maximum(hrows[q, r, :] + earows[q, r, :],
                                          0.0) + 1e-7
                        t = jnp.exp(msg - shvec)
                        tp[q, r, pl.ds(0, 16)] = t
                        tp[q, r, pl.ds(16, 16)] = t * msg

                    issue_scatters(qq, q)

                @pl.when(g2 < n_groups)
                def _():
                    issue_loads(g2, (qq + 2) % 4, q)

            return 0

        lax.fori_loop(0, iters // 4, quad_body, 0)

        for i in (iters - 2, iters - 1):
            g_i = i * _NW + wid

            @pl.when(g_i < n_groups)
            def _():
                drain_scatters(i % 4, i % 2)

        plsc.subcore_barrier()

        pltpu.sync_copy(acc.at[pl.ds(sid * zrows, zrows)], zbuf)
        pltpu.sync_copy(zbuf,
                        out_hbm.at[pl.ds(cid * npad + sid * zrows, zrows)])

    return k(h, ea_lin, src2, dst2, hmax.reshape(16), eamax_p.reshape(16))


def _layer_tail(sp_ref, h_ref, w1_ref, b1_ref, g_ref, be_ref, w2_ref, b2_ref):
    n = h_ref.shape[0]
    npad = sp_ref.shape[0] // 2
    sp = sp_ref[:n, :] + sp_ref[npad:npad + n, :]
    s = sp[:, :16]
    p = sp[:, 16:]
    den = jnp.where(s > 0, s, 1.0)
    aggr = jnp.where(s > 0, p / den, 0.0)
    hmid = h_ref[...] + aggr
    z = jnp.dot(hmid, w1_ref[...],
                preferred_element_type=jnp.float32) + b1_ref[...]
    mu = jnp.mean(z, axis=0, keepdims=True)
    var = jnp.mean((z - mu) ** 2, axis=0, keepdims=True)
    zn = (z - mu) * lax.rsqrt(var + 1e-5) * g_ref[...] + be_ref[...]
    zn = jnp.maximum(zn, 0.0)
    h2 = jnp.dot(zn, w2_ref[...],
                 preferred_element_type=jnp.float32) + b2_ref[...]
    return jnp.maximum(h2, 0.0)


def _combine1_body(sp_ref, h_ref, w1_ref, b1_ref, g_ref, be_ref,
                   w2_ref, b2_ref, out_ref, hmax_ref):
    h2 = _layer_tail(sp_ref, h_ref, w1_ref, b1_ref, g_ref, be_ref,
                     w2_ref, b2_ref)
    out_ref[...] = h2
    hmax_ref[...] = jnp.max(h2, axis=0, keepdims=True)


def _combine1(sp, h, w1, b1, g, be, w2, b2):
    n = h.shape[0]
    return pl.pallas_call(
        _combine1_body,
        out_shape=[
            jax.ShapeDtypeStruct((n, 16), jnp.float32),
            jax.ShapeDtypeStruct((1, 16), jnp.float32),
        ],
    )(sp, h, w1, b1.reshape(1, 32), g.reshape(1, 32), be.reshape(1, 32),
      w2, b2.reshape(1, 16))


def _combine2_body(sp_ref, h_ref, w1_ref, b1_ref, g_ref, be_ref,
                   w2_ref, b2_ref, fw_ref, fb_ref, out_ref):
    h2 = _layer_tail(sp_ref, h_ref, w1_ref, b1_ref, g_ref, be_ref,
                     w2_ref, b2_ref)
    logits = jnp.dot(h2, fw_ref[...],
                     preferred_element_type=jnp.float32) + fb_ref[...]
    mx = jnp.max(logits, axis=1, keepdims=True)
    lse = jnp.log(jnp.sum(jnp.exp(logits - mx), axis=1, keepdims=True)) + mx
    out_ref[...] = logits - lse


def _combine2(sp, h, w1, b1, g, be, w2, b2, fc_w, fc_b):
    n = h.shape[0]
    c = fc_w.shape[1]
    return pl.pallas_call(
        _combine2_body,
        out_shape=jax.ShapeDtypeStruct((n, c), jnp.float32),
    )(sp, h, w1, b1.reshape(1, 32), g.reshape(1, 32), be.reshape(1, 32),
      w2, b2.reshape(1, 16), fc_w, fc_b.reshape(1, c))


def kernel(x, edge_index, edge_attr, time_weights, W_node, b_node, W_edge,
           b_edge, W_time, b_time, c1_w1, c1_b1, c1_g, c1_be, c1_w2, c1_b2,
           c2_w1, c2_b1, c2_g, c2_be, c2_w2, c2_b2, fc_w, fc_b):
    src = edge_index[0]
    dst = edge_index[1]

    h0, hmax0 = _enc_nodes(x, W_node, b_node, time_weights, W_time, b_time)
    ea, eamax = _enc_edges(edge_attr, W_edge, b_edge)

    sp0 = _edge_pass(h0, ea, src, dst, hmax0, eamax)
    h1, hmax1 = _combine1(sp0, h0, c1_w1, c1_b1, c1_g, c1_be, c1_w2, c1_b2)

    sp1 = _edge_pass(h1, ea, src, dst, hmax1, eamax)
    return _combine2(sp1, h1, c2_w1, c2_b1, c2_g, c2_be, c2_w2, c2_b2,
                     fc_w, fc_b)


# speedup vs baseline: 1.4782x; 1.4782x over previous
import functools

import jax
import jax.numpy as jnp
from jax import lax
from jax.experimental import pallas as pl
from jax.experimental.pallas import tpu as pltpu
from jax.experimental.pallas import tpu_sc as plsc

_NC = 2
_NS = 16
_NW = _NC * _NS
_CHUNK = 128


def _enc_nodes_body(x_ref, wn_ref, bn_ref, t_ref, wt_ref, bt_ref,
                    h_ref, hmax_ref):
    x0 = jnp.dot(x_ref[...], wn_ref[...],
                 preferred_element_type=jnp.float32) + bn_ref[...]
    tw = t_ref[...] * wt_ref[...] + bt_ref[...]
    h = x0 * tw
    h_ref[...] = h
    hmax_ref[...] = jnp.max(h, axis=0, keepdims=True)


def _enc_nodes(x, W_node, b_node, time_weights, W_time, b_time):
    n = x.shape[0]
    return pl.pallas_call(
        _enc_nodes_body,
        out_shape=[
            jax.ShapeDtypeStruct((n, 16), jnp.float32),
            jax.ShapeDtypeStruct((1, 16), jnp.float32),
        ],
    )(x, W_node, b_node.reshape(1, 16), time_weights,
      W_time, b_time.reshape(1, 16))


def _enc_edges_body(a_hbm, w_ref, b_ref, ea_ref, emax_ref, abuf, zbuf,
                    sem_in, sem_out):
    i = pl.program_id(0)
    ng = pl.num_programs(0)
    blk = abuf.shape[1]

    def in_cp(j):
        return pltpu.make_async_copy(a_hbm.at[pl.ds(j * blk, blk)],
                                     abuf.at[j % 2], sem_in)

    def out_cp(j):
        return pltpu.make_async_copy(zbuf.at[j % 2],
                                     ea_ref.at[pl.ds(j * blk, blk)], sem_out)

    @pl.when(i == 0)
    def _():
        in_cp(0).start()

    @pl.when(i + 1 < ng)
    def _():
        in_cp(i + 1).start()

    in_cp(i).wait()

    @pl.when(i >= 2)
    def _():
        out_cp(i - 2).wait()

    z = jnp.dot(abuf[i % 2], w_ref[...],
                preferred_element_type=jnp.float32) + b_ref[...]
    zbuf[i % 2] = z
    out_cp(i).start()
    bm = jnp.max(z, axis=0, keepdims=True)

    @pl.when(i == 0)
    def _():
        emax_ref[...] = bm

    @pl.when(i > 0)
    def _():
        emax_ref[...] = jnp.maximum(emax_ref[...], bm)

    @pl.when(i == ng - 1)
    def _():
        out_cp(i - 1).wait()
        out_cp(i).wait()


def _enc_edges(edge_attr, W_edge, b_edge):
    e = edge_attr.shape[0]
    ep = e // 8
    attr_p = edge_attr.reshape(ep, 128)
    w_kron = jnp.kron(jnp.eye(8, dtype=jnp.float32), W_edge)
    b_tile = jnp.tile(b_edge, 8).reshape(1, 128)
    blk = 5000
    grid = ep // blk
    ea_p, emax_p = pl.pallas_call(
        _enc_edges_body,
        grid=(grid,),
        in_specs=[
            pl.BlockSpec(memory_space=pl.ANY),
            pl.BlockSpec((128, 128), lambda i: (0, 0)),
            pl.BlockSpec((1, 128), lambda i: (0, 0)),
        ],
        out_specs=[
            pl.BlockSpec(memory_space=pl.ANY),
            pl.BlockSpec((1, 128), lambda i: (0, 0)),
        ],
        out_shape=[
            jax.ShapeDtypeStruct((ep, 128), jnp.float32),
            jax.ShapeDtypeStruct((1, 128), jnp.float32),
        ],
        scratch_shapes=[
            pltpu.VMEM((2, blk, 128), jnp.float32),
            pltpu.VMEM((2, blk, 128), jnp.float32),
            pltpu.SemaphoreType.DMA,
            pltpu.SemaphoreType.DMA,
        ],
    )(attr_p, w_kron, b_tile)
    return ea_p, emax_p


_G = 5
_GE = _G * _CHUNK


def _edge_pass(h, ea_p, src, dst, hmax, eamax_p):
    n = h.shape[0]
    e = ea_p.shape[0] * 8
    gep = _GE // 8
    n_chunks = e // _CHUNK
    n_groups = n_chunks // _G
    iters = pl.cdiv(n_groups, _NW)
    pairs = pl.cdiv(iters, 2)
    npad = ((n + 8 * _NS - 1) // (8 * _NS)) * (8 * _NS)
    zrows = npad // _NS

    src2 = src.reshape(n_chunks, _CHUNK)
    dst2 = dst.reshape(n_chunks, _CHUNK)

    mesh = plsc.VectorSubcoreMesh(core_axis_name="c", subcore_axis_name="s")

    @functools.partial(
        pl.kernel,
        out_type=jax.ShapeDtypeStruct((_NC * npad, 32), jnp.float32),
        mesh=mesh,
        scratch_types=[
            pltpu.VMEM((4, _G, _CHUNK), jnp.int32),
            pltpu.VMEM((4, _G, _CHUNK), jnp.int32),
            pltpu.VMEM((2, _GE, 16), jnp.float32),
            pltpu.VMEM((2, _GE // 8, 128), jnp.float32),
            pltpu.VMEM((2, _GE, 32), jnp.float32),
            pltpu.VMEM((16,), jnp.float32),
            pltpu.VMEM((8, 16), jnp.float32),
            pltpu.VMEM((zrows, 32), jnp.float32),
            pltpu.VMEM_SHARED((npad, 32), jnp.float32),
            pltpu.SemaphoreType.DMA,
            pltpu.SemaphoreType.DMA,
            pltpu.SemaphoreType.DMA,
            pltpu.SemaphoreType.DMA,
            pltpu.SemaphoreType.DMA,
        ],
        compiler_params=pltpu.CompilerParams(use_tc_tiling_on_sc=False),
    )
    def k(h_hbm, ea_hbm, src_hbm, dst_hbm, hmax_hbm, eamax_hbm, out_hbm,
          sidx, didx, hrows, earows, tp, hmv, emv, zbuf, acc,
          sem_s, sem_d, sem_e, sem_g, sem_sc):
        cid = lax.axis_index("c")
        sid = lax.axis_index("s")
        wid = sid * _NC + cid

        zero16 = jnp.zeros((16,), jnp.float32)

        @plsc.parallel_loop(0, zrows, step=1, unroll=4)
        def _(i):
            zbuf[i, pl.ds(0, 16)] = zero16
            zbuf[i, pl.ds(16, 16)] = zero16

        pltpu.sync_copy(zbuf, acc.at[pl.ds(sid * zrows, zrows)])
        pltpu.sync_copy(hmax_hbm, hmv)
        pltpu.sync_copy(eamax_hbm, emv)
        em = emv[0, :]
        for kk in range(1, 8):
            em = jnp.maximum(em, emv[kk, :])
        shvec = jnp.maximum(hmv[...] + em, 0.0) + 1e-7
        plsc.subcore_barrier()

        def issue_loads(g, s4, q):
            pltpu.async_copy(src_hbm.at[pl.ds(g * _G, _G)], sidx.at[s4],
                             sem_s)
            pltpu.async_copy(dst_hbm.at[pl.ds(g * _G, _G)], didx.at[s4],
                             sem_d)
            pltpu.async_copy(ea_hbm.at[pl.ds(g * gep, gep)], earows.at[q],
                             sem_e)

        def drain_idx(g, s4):
            pltpu.make_async_copy(src_hbm.at[pl.ds(g * _G, _G)],
                                  sidx.at[s4], sem_s).wait()
            pltpu.make_async_copy(dst_hbm.at[pl.ds(g * _G, _G)],
                                  didx.at[s4], sem_d).wait()

        def issue_gathers(s4, q):
            for b in range(_G):
                pltpu.async_copy(h_hbm.at[sidx.at[s4, b]],
                                 hrows.at[q, pl.ds(b * _CHUNK, _CHUNK)],
                                 sem_g)

        def drain_gathers(s4, q):
            for b in range(_G):
                pltpu.make_async_copy(h_hbm.at[sidx.at[s4, b]],
                                      hrows.at[q, pl.ds(b * _CHUNK, _CHUNK)],
                                      sem_g).wait()

        def issue_scatters(s4, q):
            for b in range(_G):
                pltpu.async_copy(tp.at[q, pl.ds(b * _CHUNK, _CHUNK)],
                                 acc.at[didx.at[s4, b]], sem_sc, add=True)

        def drain_scatters(s4, q):
            for b in range(_G):
                pltpu.make_async_copy(tp.at[q, pl.ds(b * _CHUNK, _CHUNK)],
                                      acc.at[didx.at[s4, b]], sem_sc).wait()

        issue_loads(wid, 0, 0)
        issue_loads(_NW + wid, 1, 1)
        drain_idx(wid, 0)
        issue_gathers(0, 0)

        def quad_body(j4, _):
            for qq in range(4):
                q = qq % 2
                i = j4 * 4 + qq
                g = i * _NW + wid
                g1 = g + _NW
                g2 = g + 2 * _NW

                @pl.when(g1 < n_groups)
                def _():
                    drain_idx(g1, (qq + 1) % 4)
                    issue_gathers((qq + 1) % 4, 1 - q)

                @pl.when(g < n_groups)
                def _():
                    pltpu.make_async_copy(
                        ea_hbm.at[pl.ds(g * gep, gep)], earows.at[q],
                        sem_e).wait()
                    drain_gathers(qq, q)

                @pl.when((i >= 2) & (g - 2 * _NW < n_groups))
                def _():
                    drain_scatters((qq + 2) % 4, q)

                @pl.when(g < n_groups)
                def _():
                    @plsc.parallel_loop(0, gep, step=1, unroll=1)
                    def _(pr):
                        for u in range(8):
                            r = pr * 8 + u
                            msg = jnp.maximum(
                                hrows[q, r, :]
                                + earows[q, pr, pl.ds(u * 16, 16)],
                                0.0) + 1e-7
                            t = jnp.exp(msg - shvec)
                            tp[q, r, pl.ds(0, 16)] = t
                            tp[q, r, pl.ds(16, 16)] = t * msg

                    issue_scatters(qq, q)

                @pl.when(g2 < n_groups)
                def _():
                    issue_loads(g2, (qq + 2) % 4, q)

            return 0

        lax.fori_loop(0, iters // 4, quad_body, 0)

        for i in (iters - 2, iters - 1):
            g_i = i * _NW + wid

            @pl.when(g_i < n_groups)
            def _():
                drain_scatters(i % 4, i % 2)

        plsc.subcore_barrier()

        pltpu.sync_copy(acc.at[pl.ds(sid * zrows, zrows)], zbuf)
        pltpu.sync_copy(zbuf,
                        out_hbm.at[pl.ds(cid * npad + sid * zrows, zrows)])

    return k(h, ea_p, src2, dst2, hmax.reshape(16), eamax_p.reshape(8, 16))


def _layer_tail(sp_ref, h_ref, w1_ref, b1_ref, g_ref, be_ref, w2_ref, b2_ref):
    n = h_ref.shape[0]
    npad = sp_ref.shape[0] // 2
    sp = sp_ref[:n, :] + sp_ref[npad:npad + n, :]
    s = sp[:, :16]
    p = sp[:, 16:]
    den = jnp.where(s > 0, s, 1.0)
    aggr = jnp.where(s > 0, p / den, 0.0)
    hmid = h_ref[...] + aggr
    z = jnp.dot(hmid, w1_ref[...],
                preferred_element_type=jnp.float32) + b1_ref[...]
    mu = jnp.mean(z, axis=0, keepdims=True)
    var = jnp.mean((z - mu) ** 2, axis=0, keepdims=True)
    zn = (z - mu) * lax.rsqrt(var + 1e-5) * g_ref[...] + be_ref[...]
    zn = jnp.maximum(zn, 0.0)
    h2 = jnp.dot(zn, w2_ref[...],
                 preferred_element_type=jnp.float32) + b2_ref[...]
    return jnp.maximum(h2, 0.0)


def _combine1_body(sp_ref, h_ref, w1_ref, b1_ref, g_ref, be_ref,
                   w2_ref, b2_ref, out_ref, hmax_ref):
    h2 = _layer_tail(sp_ref, h_ref, w1_ref, b1_ref, g_ref, be_ref,
                     w2_ref, b2_ref)
    out_ref[...] = h2
    hmax_ref[...] = jnp.max(h2, axis=0, keepdims=True)


def _combine1(sp, h, w1, b1, g, be, w2, b2):
    n = h.shape[0]
    return pl.pallas_call(
        _combine1_body,
        out_shape=[
            jax.ShapeDtypeStruct((n, 16), jnp.float32),
            jax.ShapeDtypeStruct((1, 16), jnp.float32),
        ],
    )(sp, h, w1, b1.reshape(1, 32), g.reshape(1, 32), be.reshape(1, 32),
      w2, b2.reshape(1, 16))


def _combine2_body(sp_ref, h_ref, w1_ref, b1_ref, g_ref, be_ref,
                   w2_ref, b2_ref, fw_ref, fb_ref, out_ref):
    h2 = _layer_tail(sp_ref, h_ref, w1_ref, b1_ref, g_ref, be_ref,
                     w2_ref, b2_ref)
    logits = jnp.dot(h2, fw_ref[...],
                     preferred_element_type=jnp.float32) + fb_ref[...]
    mx = jnp.max(logits, axis=1, keepdims=True)
    lse = jnp.log(jnp.sum(jnp.exp(logits - mx), axis=1, keepdims=True)) + mx
    out_ref[...] = logits - lse


def _combine2(sp, h, w1, b1, g, be, w2, b2, fc_w, fc_b):
    n = h.shape[0]
    c = fc_w.shape[1]
    return pl.pallas_call(
        _combine2_body,
        out_shape=jax.ShapeDtypeStruct((n, c), jnp.float32),
    )(sp, h, w1, b1.reshape(1, 32), g.reshape(1, 32), be.reshape(1, 32),
      w2, b2.reshape(1, 16), fc_w, fc_b.reshape(1, c))


def kernel(x, edge_index, edge_attr, time_weights, W_node, b_node, W_edge,
           b_edge, W_time, b_time, c1_w1, c1_b1, c1_g, c1_be, c1_w2, c1_b2,
           c2_w1, c2_b1, c2_g, c2_be, c2_w2, c2_b2, fc_w, fc_b):
    src = edge_index[0]
    dst = edge_index[1]

    h0, hmax0 = _enc_nodes(x, W_node, b_node, time_weights, W_time, b_time)
    ea, eamax = _enc_edges(edge_attr, W_edge, b_edge)

    sp0 = _edge_pass(h0, ea, src, dst, hmax0, eamax)
    h1, hmax1 = _combine1(sp0, h0, c1_w1, c1_b1, c1_g, c1_be, c1_w2, c1_b2)

    sp1 = _edge_pass(h1, ea, src, dst, hmax1, eamax)
    return _combine2(sp1, h1, c2_w1, c2_b1, c2_g, c2_be, c2_w2, c2_b2,
                     fc_w, fc_b)
